# baseline (speedup 1.0000x reference)
import functools

import jax
import jax.numpy as jnp
from jax import lax
from jax.experimental import pallas as pl
from jax.experimental.pallas import tpu as pltpu
from jax.experimental.pallas import tpu_sc as plsc

L = 16
NC = 2
NS = 16
NW = NC * NS
CH = 128


def _s_body(x_ref, w_ref, b_ref, o_ref):
    s = jnp.dot(x_ref[...], w_ref[...], preferred_element_type=jnp.float32)
    col = lax.broadcasted_iota(jnp.int32, s.shape, 1)
    o_ref[...] = s + jnp.where(col == 1, b_ref[0, 0], 0.0)


def _s_precompute(x, w_packed, b):
    n, d = x.shape
    blk = 1000
    return pl.pallas_call(
        _s_body,
        grid=(n // blk,),
        in_specs=[
            pl.BlockSpec((blk, d), lambda i: (i, 0)),
            pl.BlockSpec((d, 8), lambda i: (0, 0)),
            pl.BlockSpec(memory_space=pltpu.SMEM),
        ],
        out_specs=pl.BlockSpec((blk, 8), lambda i: (i, 0)),
        out_shape=jax.ShapeDtypeStruct((n, 8), jnp.float32),
    )(x, w_packed, b)


def _sc_edge_body(x_hbm, edge_hbm, s1_hbm, s2_hbm, out_hbm,
                  idx_v, rows_v, s1b, s2b, acc_sh,
                  isem0, isem1, isem2,
                  gsem0, gsem1, gsem2,
                  ssem0, ssem1, ssem2,
                  n, d, base_cpt, n_extra):
    cid = lax.axis_index("c")
    sid = lax.axis_index("s")
    wid = sid * NC + cid
    zrows = 80
    n_units = n // zrows
    isems = (isem0, isem1, isem2)
    gsems = (gsem0, gsem1, gsem2)
    ssems = (ssem0, ssem1, ssem2)
    cpt = base_cpt + jnp.where(wid < n_extra, 1, 0)
    tail0 = NW * base_cpt * CH

    def _off(c):
        return jnp.where(c < base_cpt,
                         (wid * base_cpt + c) * CH, tail0 + wid * CH)

    def _idx_start(c, sem):
        pltpu.async_copy(edge_hbm.at[pl.ds(0, 2), pl.ds(_off(c), CH)],
                         idx_v.at[lax.rem(c, 4)], sem)

    def _idx_wait(c, sem):
        pltpu.make_async_copy(edge_hbm.at[pl.ds(0, 2), pl.ds(_off(c), CH)],
                              idx_v.at[lax.rem(c, 4)], sem).wait()

    def _gathers_start(c, p, sem):
        m = lax.rem(c, 4)
        pltpu.async_copy(x_hbm.at[idx_v.at[m, 0]], rows_v.at[p], sem)
        pltpu.async_copy(s1_hbm.at[idx_v.at[m, 0]], s1b.at[lax.rem(c, 2)], sem)
        pltpu.async_copy(s2_hbm.at[idx_v.at[m, 1]], s2b.at[lax.rem(c, 2)], sem)

    def _gathers_wait(c, p, sem):
        m = lax.rem(c, 4)
        pltpu.make_async_copy(x_hbm.at[idx_v.at[m, 0]], rows_v.at[p],
                              sem).wait()
        pltpu.make_async_copy(s1_hbm.at[idx_v.at[m, 0]],
                              s1b.at[lax.rem(c, 2)], sem).wait()
        pltpu.make_async_copy(s2_hbm.at[idx_v.at[m, 1]],
                              s2b.at[lax.rem(c, 2)], sem).wait()

    def _scatter_start(c, p, sem):
        m = lax.rem(c, 4)
        pltpu.async_copy(rows_v.at[p], acc_sh.at[idx_v.at[m, 1]], sem,
                         add=True)

    def _scatter_wait(c, p, sem):
        m = lax.rem(c, 4)
        pltpu.make_async_copy(rows_v.at[p], acc_sh.at[idx_v.at[m, 1]],
                              sem).wait()

    _idx_start(0, isems[0])
    _idx_start(1, isems[1])

    def _zrow(r, carry):
        for k in range(d // L):
            rows_v[0, r, pl.ds(k * L, L)] = jnp.zeros((L,), jnp.float32)
        return carry
    lax.fori_loop(0, zrows, _zrow, 0)

    def _zunit(k, carry):
        u = sid + NS * k

        @pl.when(u < n_units)
        def _():
            pltpu.sync_copy(rows_v.at[0, pl.ds(0, zrows)],
                            acc_sh.at[pl.ds(u * zrows, zrows)])
        return carry
    lax.fori_loop(0, (n_units + NS - 1) // NS, _zunit, 0)
    _idx_wait(0, isems[0])
    _gathers_start(0, 0, gsems[0])
    plsc.subcore_barrier()

    def _compute(c, p):
        m2 = lax.rem(c, 2)
        for g in range(CH // L):
            z = s1b[m2, pl.ds(g * L, L)] + s2b[m2, pl.ds(g * L, L)]
            ez = jnp.exp(-2.0 * jnp.abs(z))
            t = (1.0 - ez) / (1.0 + ez)
            alpha = jnp.where(z < 0.0, -t, t)

            @plsc.parallel_loop(0, L, unroll=8)
            def _edge(j, _alpha=alpha, _g=g):
                aj = _alpha.at[jnp.full((L,), j, jnp.int32)].get(
                    mode="promise_in_bounds")
                row = _g * L + j
                for k in range(d // L):
                    rows_v[p, row, pl.ds(k * L, L)] = (
                        rows_v[p, row, pl.ds(k * L, L)] * aj)

    def _third(c, p):
        p1 = (p + 1) % 3
        p2 = (p + 2) % 3

        @pl.when(c + 1 < cpt)
        def _():
            _idx_wait(c + 1, isems[p1])

        @pl.when(c - 2 >= 0)
        def _():
            _scatter_wait(c - 2, p1, ssems[p1])

        @pl.when(c + 1 < cpt)
        def _():
            _gathers_start(c + 1, p1, gsems[p1])
        _gathers_wait(c, p, gsems[p])
        _compute(c, p)
        _scatter_start(c, p, ssems[p])

        @pl.when(c + 2 < cpt)
        def _():
            _idx_start(c + 2, isems[p2])

    def _triple(c3, carry):
        c = 3 * c3
        _third(c, 0)
        for p in (1, 2):
            @pl.when(c + p < cpt)
            def _(_p=p):
                _third(c + _p, _p)
        return carry

    lax.fori_loop(0, (cpt + 2) // 3, _triple, 0)
    cpt_hi = base_cpt + 1

    @pl.when(wid < n_extra)
    def _():
        _scatter_wait(cpt_hi - 1, (cpt_hi - 1) % 3, ssems[(cpt_hi - 1) % 3])
        _scatter_wait(cpt_hi - 2, (cpt_hi - 2) % 3, ssems[(cpt_hi - 2) % 3])

    @pl.when(wid >= n_extra)
    def _():
        _scatter_wait(base_cpt - 1, (base_cpt - 1) % 3,
                      ssems[(base_cpt - 1) % 3])
        _scatter_wait(base_cpt - 2, (base_cpt - 2) % 3,
                      ssems[(base_cpt - 2) % 3])
    plsc.subcore_barrier()

    def _eunit(k, carry):
        u = sid + NS * k

        @pl.when(u < n_units)
        def _():
            pltpu.sync_copy(acc_sh.at[pl.ds(u * zrows, zrows)],
                            out_hbm.at[pl.ds(cid * n + u * zrows, zrows)])
        return carry
    lax.fori_loop(0, (n_units + NS - 1) // NS, _eunit, 0)


def _sc_edge_stage(x, edge_index, s1, s2):
    n, d = x.shape
    e = edge_index.shape[1]
    base_cpt = e // (NW * CH)
    n_extra = (e - NW * base_cpt * CH) // CH
    mesh = plsc.VectorSubcoreMesh(core_axis_name="c", subcore_axis_name="s")
    body = functools.partial(_sc_edge_body, n=n, d=d, base_cpt=base_cpt,
                             n_extra=n_extra)
    return pl.kernel(
        body,
        out_type=jax.ShapeDtypeStruct((NC * n, d), jnp.float32),
        mesh=mesh,
        compiler_params=pltpu.CompilerParams(needs_layout_passes=False),
        scratch_types=(
            [
                pltpu.VMEM((4, 2, CH), jnp.int32),
                pltpu.VMEM((3, CH, d), jnp.float32),
                pltpu.VMEM((2, CH), jnp.float32),
                pltpu.VMEM((2, CH), jnp.float32),
                pltpu.VMEM_SHARED((n, d), jnp.float32),
            ]
            + [pltpu.SemaphoreType.DMA] * 9
        ),
    )(x, edge_index, s1, s2)


def _combine_body(x_ref, p0_ref, p1_ref, eps_ref, o_ref):
    eps = eps_ref[0, 0]
    o_ref[...] = eps * x_ref[...] + (1.0 - eps) * (p0_ref[...] + p1_ref[...])


def _combine(x, partial, eps):
    n, d = x.shape
    blk = 1000
    nb = n // blk
    return pl.pallas_call(
        _combine_body,
        grid=(nb,),
        in_specs=[
            pl.BlockSpec((blk, d), lambda i: (i, 0)),
            pl.BlockSpec((blk, d), lambda i: (i, 0)),
            pl.BlockSpec((blk, d), lambda i, _nb=nb: (i + _nb, 0)),
            pl.BlockSpec(memory_space=pltpu.SMEM),
        ],
        out_specs=pl.BlockSpec((blk, d), lambda i: (i, 0)),
        out_shape=jax.ShapeDtypeStruct((n, d), jnp.float32),
    )(x, partial, partial, eps)


def kernel(x, edge_index, att_w, att_b, eps):
    n, d = x.shape
    e = edge_index.shape[1]
    w2 = att_w.reshape(2, d).T
    w_packed = jnp.pad(w2, ((0, 0), (0, 6)))
    b = att_b.reshape(1, 1)
    s8 = _s_precompute(x, w_packed, b)
    s1 = s8[:, 0]
    s2 = s8[:, 1]
    partial = _sc_edge_stage(x, edge_index, s1, s2)
    eps_arr = jnp.asarray(eps, jnp.float32).reshape(1, 1)
    return _combine(x, partial, eps_arr)
